# Initial kernel scaffold; baseline (speedup 1.0000x reference)
import jax
import jax.numpy as jnp
from jax.experimental import pallas as pl


def kernel(emb, W1, b1, W2, b2, Wr1, root1, bias1, Wr2, root2, bias2, edge_index, edge_type):
    raise NotImplementedError("write your pallas kernel here")


# speedup vs baseline: 17.0788x; 17.0788x over previous
import functools

import jax
import jax.numpy as jnp
from jax import lax
from jax.experimental import pallas as pl
from jax.experimental.pallas import tpu as pltpu
from jax.experimental.pallas import tpu_sc as plsc

N = 50000
E = 800000
R = 8
EMB = 64
HID = 64
LABELS = 16
OUT_F = 112
NR = N * R

NC = 2
NS = 16
L = 16

CH = 512
SB = 128
NPT = N // NS
CPT = NR // NS

f32 = jnp.float32
i32 = jnp.int32


def _fill(ref, n, val):
  def body(i, _):
    ref[pl.ds(i * L, L)] = jnp.full((L,), val, f32)
    return 0
  lax.fori_loop(0, n // L, body, 0)
  if n % L:
    ref[pl.ds(n - L, L)] = jnp.full((L,), val, f32)


def _counts_body(dst_h, et_h, out_h, dstf, etf, compq, ones, cbuf, acc):
  c = lax.axis_index("c")
  s = lax.axis_index("s")
  m = E // NC
  n_full = m // CH
  tail = m - n_full * CH
  base0 = c * m

  _fill(cbuf, CPT, 0.0)
  pltpu.sync_copy(cbuf, acc.at[pl.ds(s * CPT, CPT)])
  _fill(ones, SB, 1.0)
  plsc.subcore_barrier()

  def chunk(clen, base):
    nb = clen // SB
    pltpu.sync_copy(dst_h.at[pl.ds(base, clen)], dstf.at[pl.ds(0, clen)])
    pltpu.sync_copy(et_h.at[pl.ds(base, clen)], etf.at[pl.ds(0, clen)])
    for j in range(nb):
      def cb(i2, _):
        sl = pl.ds(j * SB + i2 * L, L)
        compq[j, pl.ds(i2 * L, L)] = dstf[sl] * R + etf[sl]
        return 0
      lax.fori_loop(0, SB // L, cb, 0)
    for j in range(nb):
      pltpu.sync_copy(ones, acc.at[compq.at[j]], add=True)

  my = (n_full + NS - 1 - s) // NS
  def loop_body(k, _):
    chunk(CH, base0 + (s + k * NS) * CH)
    return 0
  lax.fori_loop(0, my, loop_body, 0)

  @pl.when(s == NS - 1)
  def _():
    chunk(tail, base0 + n_full * CH)

  plsc.subcore_barrier()
  pltpu.sync_copy(acc.at[pl.ds(s * CPT, CPT)], cbuf)
  pltpu.sync_copy(cbuf, out_h.at[pl.ds(c * NR + s * CPT, CPT)])


def _make_counts():
  mesh = plsc.VectorSubcoreMesh(core_axis_name="c", subcore_axis_name="s")
  return pl.kernel(
      _counts_body,
      out_type=jax.ShapeDtypeStruct((NC * NR,), f32),
      mesh=mesh,
      compiler_params=pltpu.CompilerParams(use_tc_tiling_on_sc=False),
      scratch_types=[
          pltpu.VMEM((CH,), i32),
          pltpu.VMEM((CH,), i32),
          pltpu.VMEM((CH // SB, SB), i32),
          pltpu.VMEM((SB,), f32),
          pltpu.VMEM((CPT,), f32),
          pltpu.VMEM_SHARED((NR,), f32),
      ],
  )


def _layer_body(width, split_edges, tab_h, inv_h, src_h, dst_h, et_h,
                out_h, srcf, dstf, etf, idxf, compf, dstq, invv, msgs,
                sem, sem2, acc):
  c = lax.axis_index("c")
  s = lax.axis_index("s")
  m = E // NC if split_edges else E
  n_full = m // CH
  tail = m - n_full * CH
  base0 = c * m if split_edges else 0

  def zb(g, _):
    for w in range(width // L):
      msgs[g, pl.ds(w * L, L)] = jnp.zeros((L,), f32)
    return 0
  lax.fori_loop(0, CH, zb, 0)
  for k in range(NPT // CH):
    pltpu.sync_copy(msgs, acc.at[pl.ds(s * NPT + k * CH, CH)])
  ntl = NPT - (NPT // CH) * CH
  pltpu.sync_copy(msgs.at[pl.ds(0, ntl)],
                  acc.at[pl.ds(s * NPT + (NPT // CH) * CH, ntl)])
  plsc.subcore_barrier()

  def chunk(clen, base):
    nb = clen // SB
    pltpu.sync_copy(src_h.at[pl.ds(base, clen)], srcf.at[pl.ds(0, clen)])
    pltpu.sync_copy(dst_h.at[pl.ds(base, clen)], dstf.at[pl.ds(0, clen)])
    pltpu.sync_copy(et_h.at[pl.ds(base, clen)], etf.at[pl.ds(0, clen)])
    for j in range(nb):
      def cb(i2, _):
        sl = pl.ds(j * SB + i2 * L, L)
        sv = srcf[sl]
        dv = dstf[sl]
        tv = etf[sl]
        idxf[sl] = sv * R + tv
        compf[sl] = dv * R + tv
        dstq[j, pl.ds(i2 * L, L)] = dv
        return 0
      lax.fori_loop(0, SB // L, cb, 0)
    if split_edges:
      gsrc = tab_h.at[idxf.at[pl.ds(0, clen)]]
    else:
      gsrc = tab_h.at[c].at[idxf.at[pl.ds(0, clen)]]
    cp1 = pltpu.async_copy(gsrc, msgs.at[pl.ds(0, clen)], sem)
    cp2 = pltpu.async_copy(inv_h.at[compf.at[pl.ds(0, clen)]],
                           invv.at[pl.ds(0, clen)], sem2)
    cp1.wait()
    cp2.wait()

    def sb_body(g, _):
      vinv = invv[pl.ds(g * L, L)]
      for j in range(L):
        e = g * L + j
        vv = jnp.broadcast_to(vinv[j], (L,))
        for w in range(width // L):
          cur = msgs[e, pl.ds(w * L, L)]
          msgs[e, pl.ds(w * L, L)] = cur * vv
      return 0
    lax.fori_loop(0, clen // L, sb_body, 0)

    for j in range(nb):
      pltpu.sync_copy(msgs.at[pl.ds(j * SB, SB)], acc.at[dstq.at[j]],
                      add=True)

  my = (n_full + NS - 1 - s) // NS
  def loop_body(k, _):
    chunk(CH, base0 + (s + k * NS) * CH)
    return 0
  lax.fori_loop(0, my, loop_body, 0)

  @pl.when(s == NS - 1)
  def _():
    chunk(tail, base0 + n_full * CH)

  plsc.subcore_barrier()
  for k in range(NPT // CH):
    pltpu.sync_copy(acc.at[pl.ds(s * NPT + k * CH, CH)], msgs)
    pltpu.sync_copy(msgs, out_h.at[c, pl.ds(s * NPT + k * CH, CH)])
  pltpu.sync_copy(acc.at[pl.ds(s * NPT + (NPT // CH) * CH, ntl)],
                  msgs.at[pl.ds(0, ntl)])
  pltpu.sync_copy(msgs.at[pl.ds(0, ntl)],
                  out_h.at[c, pl.ds(s * NPT + (NPT // CH) * CH, ntl)])


def _make_layer(width, split_edges):
  mesh = plsc.VectorSubcoreMesh(core_axis_name="c", subcore_axis_name="s")
  return pl.kernel(
      functools.partial(_layer_body, width, split_edges),
      out_type=jax.ShapeDtypeStruct((NC, N, width), f32),
      mesh=mesh,
      compiler_params=pltpu.CompilerParams(use_tc_tiling_on_sc=False),
      scratch_types=[
          pltpu.VMEM((CH,), i32),
          pltpu.VMEM((CH,), i32),
          pltpu.VMEM((CH,), i32),
          pltpu.VMEM((CH,), i32),
          pltpu.VMEM((CH,), i32),
          pltpu.VMEM((CH // SB, SB), i32),
          pltpu.VMEM((CH,), f32),
          pltpu.VMEM((CH, width), f32),
          pltpu.SemaphoreType.DMA,
          pltpu.SemaphoreType.DMA,
          pltpu.VMEM_SHARED((N, width), f32),
      ],
  )


BLK = 2000


def _stage_b_kern(emb_r, w1_r, b1_r, w2_r, b2_r, wcat_r, root1_r, bias1_r,
                  t1_r, r1_r):
  x1 = jax.nn.sigmoid(
      jnp.dot(emb_r[...], w1_r[...], preferred_element_type=f32) + b1_r[...])
  x = jax.nn.sigmoid(
      jnp.dot(x1, w2_r[...], preferred_element_type=f32) + b2_r[...])
  t = jnp.dot(x, wcat_r[...], preferred_element_type=f32)
  t1_r[0] = t[:, :R * 32]
  t1_r[1] = t[:, R * 32:]
  r1_r[...] = jnp.dot(x, root1_r[...], preferred_element_type=f32) + bias1_r[...]


def _inv_kern(c_r, inv_r):
  cnt = c_r[0] + c_r[1]
  inv_r[...] = jnp.where(cnt > 0, 1.0 / jnp.maximum(cnt, 1.0), 0.0)


def _stage_d_kern(agg_r, r1_r, wcat2_r, root2_r, bias2_r, t2_r, r2_r):
  h = jnp.maximum(
      jnp.concatenate([agg_r[0], agg_r[1]], axis=1) + r1_r[...], 0.0)
  t2_r[...] = jnp.dot(h, wcat2_r[...], preferred_element_type=f32)
  r2_r[...] = jnp.dot(h, root2_r[...], preferred_element_type=f32) + bias2_r[...]


def _stage_f_kern(agg_r, r2_r, out_r):
  out_r[...] = jax.nn.sigmoid(agg_r[0] + agg_r[1] + r2_r[...])


def _full(shape):
  return pl.BlockSpec(shape, lambda i: tuple(0 for _ in shape))


def kernel(emb, W1, b1, W2, b2, Wr1, root1, bias1, Wr2, root2, bias2,
           edge_index, edge_type):
  src = edge_index[0]
  dst = edge_index[1]
  et = edge_type

  w1t = W1.T
  w2t = W2.T
  b1r = b1.reshape(1, OUT_F)
  b2r = b2.reshape(1, EMB)
  wlo = Wr1[:, :, :32].transpose(1, 0, 2).reshape(EMB, R * 32)
  whi = Wr1[:, :, 32:].transpose(1, 0, 2).reshape(EMB, R * 32)
  wcat = jnp.concatenate([wlo, whi], axis=1)
  wcat2 = Wr2.transpose(1, 0, 2).reshape(HID, R * LABELS)
  bias1r = bias1.reshape(1, HID)
  bias2r = bias2.reshape(1, LABELS)
  counts = _make_counts()(dst, et)

  inv2d = pl.pallas_call(
      _inv_kern,
      grid=(1,),
      in_specs=[pl.BlockSpec((NC, 3125, 128), lambda i: (0, 0, 0))],
      out_specs=pl.BlockSpec((3125, 128), lambda i: (0, 0)),
      out_shape=jax.ShapeDtypeStruct((3125, 128), f32),
  )(counts.reshape(NC, 3125, 128))
  inv = inv2d.reshape(NR)

  t1, r1 = pl.pallas_call(
      _stage_b_kern,
      grid=(N // BLK,),
      in_specs=[
          pl.BlockSpec((BLK, EMB), lambda i: (i, 0)),
          _full((EMB, OUT_F)),
          _full((1, OUT_F)),
          _full((OUT_F, EMB)),
          _full((1, EMB)),
          _full((EMB, 512)),
          _full((EMB, HID)),
          _full((1, HID)),
      ],
      out_specs=[
          pl.BlockSpec((NC, BLK, R * 32), lambda i: (0, i, 0)),
          pl.BlockSpec((BLK, HID), lambda i: (i, 0)),
      ],
      out_shape=[
          jax.ShapeDtypeStruct((NC, N, R * 32), f32),
          jax.ShapeDtypeStruct((N, HID), f32),
      ],
  )(emb, w1t, b1r, w2t, b2r, wcat, root1, bias1r)

  agg1 = _make_layer(32, False)(
      t1.reshape(NC, NR, 32), inv, src, dst, et)

  t2, r2 = pl.pallas_call(
      _stage_d_kern,
      grid=(N // BLK,),
      in_specs=[
          pl.BlockSpec((NC, BLK, 32), lambda i: (0, i, 0)),
          pl.BlockSpec((BLK, HID), lambda i: (i, 0)),
          _full((HID, R * LABELS)),
          _full((HID, LABELS)),
          _full((1, LABELS)),
      ],
      out_specs=[
          pl.BlockSpec((BLK, R * LABELS), lambda i: (i, 0)),
          pl.BlockSpec((BLK, LABELS), lambda i: (i, 0)),
      ],
      out_shape=[
          jax.ShapeDtypeStruct((N, R * LABELS), f32),
          jax.ShapeDtypeStruct((N, LABELS), f32),
      ],
  )(agg1, r1, wcat2, root2, bias2r)

  agg2 = _make_layer(LABELS, True)(
      t2.reshape(NR, LABELS), inv, src, dst, et)

  out = pl.pallas_call(
      _stage_f_kern,
      grid=(N // BLK,),
      in_specs=[
          pl.BlockSpec((NC, BLK, LABELS), lambda i: (0, i, 0)),
          pl.BlockSpec((BLK, LABELS), lambda i: (i, 0)),
      ],
      out_specs=pl.BlockSpec((BLK, LABELS), lambda i: (i, 0)),
      out_shape=jax.ShapeDtypeStruct((N, LABELS), f32),
  )(agg2, r2)
  return out


# speedup vs baseline: 22.3308x; 1.3075x over previous
import functools

import jax
import jax.numpy as jnp
from jax import lax
from jax.experimental import pallas as pl
from jax.experimental.pallas import tpu as pltpu
from jax.experimental.pallas import tpu_sc as plsc

N = 50000
E = 800000
R = 8
EMB = 64
HID = 64
LABELS = 16
OUT_F = 112
NR = N * R

NC = 2
NS = 16
L = 16

CH = 512
SB = 128
NPT = N // NS
CPT = NR // NS

f32 = jnp.float32
i32 = jnp.int32


def _fill(ref, n, val):
  def body(i, _):
    ref[pl.ds(i * L, L)] = jnp.full((L,), val, f32)
    return 0
  lax.fori_loop(0, n // L, body, 0)
  if n % L:
    ref[pl.ds(n - L, L)] = jnp.full((L,), val, f32)


def _counts_body(dst_h, et_h, out_h, dstf, etf, compq, ones, cbuf, acc):
  c = lax.axis_index("c")
  s = lax.axis_index("s")
  m = E // NC
  n_full = m // CH
  tail = m - n_full * CH
  base0 = c * m

  _fill(cbuf, CPT, 0.0)
  pltpu.sync_copy(cbuf, acc.at[pl.ds(s * CPT, CPT)])
  _fill(ones, SB, 1.0)
  plsc.subcore_barrier()

  def chunk(clen, base):
    nb = clen // SB
    pltpu.sync_copy(dst_h.at[pl.ds(base, clen)], dstf.at[pl.ds(0, clen)])
    pltpu.sync_copy(et_h.at[pl.ds(base, clen)], etf.at[pl.ds(0, clen)])
    for j in range(nb):
      def cb(i2, _):
        sl = pl.ds(j * SB + i2 * L, L)
        compq[j, pl.ds(i2 * L, L)] = dstf[sl] * R + etf[sl]
        return 0
      lax.fori_loop(0, SB // L, cb, 0)
    for j in range(nb):
      pltpu.sync_copy(ones, acc.at[compq.at[j]], add=True)

  my = (n_full + NS - 1 - s) // NS
  def loop_body(k, _):
    chunk(CH, base0 + (s + k * NS) * CH)
    return 0
  lax.fori_loop(0, my, loop_body, 0)

  @pl.when(s == NS - 1)
  def _():
    chunk(tail, base0 + n_full * CH)

  plsc.subcore_barrier()
  pltpu.sync_copy(acc.at[pl.ds(s * CPT, CPT)], cbuf)
  pltpu.sync_copy(cbuf, out_h.at[pl.ds(c * NR + s * CPT, CPT)])


def _make_counts():
  mesh = plsc.VectorSubcoreMesh(core_axis_name="c", subcore_axis_name="s")
  return pl.kernel(
      _counts_body,
      out_type=jax.ShapeDtypeStruct((NC * NR,), f32),
      mesh=mesh,
      compiler_params=pltpu.CompilerParams(use_tc_tiling_on_sc=False),
      scratch_types=[
          pltpu.VMEM((CH,), i32),
          pltpu.VMEM((CH,), i32),
          pltpu.VMEM((CH // SB, SB), i32),
          pltpu.VMEM((SB,), f32),
          pltpu.VMEM((CPT,), f32),
          pltpu.VMEM_SHARED((NR,), f32),
      ],
  )


def _layer_body(width, split_edges, tab_h, inv_h, src_h, dst_h, et_h,
                out_h, srcf, dstf, etf, srcg, dstg, etg, idxf, compf, dstq,
                invv, msgs, sema0, sema1, sem, sem2, sem3, acc):
  c = lax.axis_index("c")
  s = lax.axis_index("s")
  m = E // NC if split_edges else E
  n_full = m // CH
  tail = m - n_full * CH
  base0 = c * m if split_edges else 0
  ebufs = ((srcf, dstf, etf, sema0), (srcg, dstg, etg, sema1))

  def start_load(b, base):
    sf, df, tf, sm = ebufs[b]
    pltpu.async_copy(src_h.at[pl.ds(base, CH)], sf, sm)
    pltpu.async_copy(dst_h.at[pl.ds(base, CH)], df, sm)
    pltpu.async_copy(et_h.at[pl.ds(base, CH)], tf, sm)

  def wait_load(b):
    sf, df, tf, sm = ebufs[b]
    pltpu.make_async_copy(src_h.at[pl.ds(0, CH)], sf, sm).wait()
    pltpu.make_async_copy(dst_h.at[pl.ds(0, CH)], df, sm).wait()
    pltpu.make_async_copy(et_h.at[pl.ds(0, CH)], tf, sm).wait()

  def drain_scatters():
    for j in range(CH // SB):
      pltpu.make_async_copy(msgs.at[pl.ds(j * SB, SB)], acc.at[dstq.at[j]],
                            sem3).wait()

  def zb(g, _):
    for w in range(width // L):
      msgs[g, pl.ds(w * L, L)] = jnp.zeros((L,), f32)
    return 0
  lax.fori_loop(0, CH, zb, 0)
  for k in range(NPT // CH):
    pltpu.sync_copy(msgs, acc.at[pl.ds(s * NPT + k * CH, CH)])
  ntl = NPT - (NPT // CH) * CH
  pltpu.sync_copy(msgs.at[pl.ds(0, ntl)],
                  acc.at[pl.ds(s * NPT + (NPT // CH) * CH, ntl)])
  plsc.subcore_barrier()

  def compute_idx(b, clen):
    sf, df, tf, _ = ebufs[b]
    for j in range(clen // SB):
      def cb(i2, _):
        sl = pl.ds(j * SB + i2 * L, L)
        sv = sf[sl]
        dv = df[sl]
        tv = tf[sl]
        idxf[sl] = sv * R + tv
        compf[sl] = dv * R + tv
        dstq[j, pl.ds(i2 * L, L)] = dv
        return 0
      lax.fori_loop(0, SB // L, cb, 0)

  def gather_scale(clen):
    if split_edges:
      gsrc = tab_h.at[idxf.at[pl.ds(0, clen)]]
    else:
      gsrc = tab_h.at[c].at[idxf.at[pl.ds(0, clen)]]
    cp1 = pltpu.async_copy(gsrc, msgs.at[pl.ds(0, clen)], sem)
    cp2 = pltpu.async_copy(inv_h.at[compf.at[pl.ds(0, clen)]],
                           invv.at[pl.ds(0, clen)], sem2)
    cp1.wait()
    cp2.wait()

    def sb_body(g, _):
      vinv = invv[pl.ds(g * L, L)]
      for j in range(L):
        e = g * L + j
        vv = jnp.broadcast_to(vinv[j], (L,))
        for w in range(width // L):
          cur = msgs[e, pl.ds(w * L, L)]
          msgs[e, pl.ds(w * L, L)] = cur * vv
      return 0
    lax.fori_loop(0, clen // L, sb_body, 0)

  my = (n_full + NS - 1 - s) // NS
  def base_k(k):
    return base0 + (s + k * NS) * CH

  @pl.when(my > 0)
  def _():
    start_load(0, base_k(0))

  def loop_body(k, _):
    def do(b):
      wait_load(b)
      @pl.when(k + 1 < my)
      def _():
        start_load(1 - b, base_k(k + 1))
      @pl.when(k > 0)
      def _():
        drain_scatters()
      compute_idx(b, CH)
      gather_scale(CH)
      for j in range(CH // SB):
        pltpu.async_copy(msgs.at[pl.ds(j * SB, SB)], acc.at[dstq.at[j]],
                         sem3, add=True)
    @pl.when(k % 2 == 0)
    def _():
      do(0)
    @pl.when(k % 2 == 1)
    def _():
      do(1)
    return 0
  lax.fori_loop(0, my, loop_body, 0)
  @pl.when(my > 0)
  def _():
    drain_scatters()

  @pl.when(s == NS - 1)
  def _():
    tb = base0 + n_full * CH
    pltpu.sync_copy(src_h.at[pl.ds(tb, tail)], srcf.at[pl.ds(0, tail)])
    pltpu.sync_copy(dst_h.at[pl.ds(tb, tail)], dstf.at[pl.ds(0, tail)])
    pltpu.sync_copy(et_h.at[pl.ds(tb, tail)], etf.at[pl.ds(0, tail)])
    compute_idx(0, tail)
    gather_scale(tail)
    for j in range(tail // SB):
      pltpu.sync_copy(msgs.at[pl.ds(j * SB, SB)], acc.at[dstq.at[j]],
                      add=True)

  plsc.subcore_barrier()
  for k in range(NPT // CH):
    pltpu.sync_copy(acc.at[pl.ds(s * NPT + k * CH, CH)], msgs)
    pltpu.sync_copy(msgs, out_h.at[c, pl.ds(s * NPT + k * CH, CH)])
  pltpu.sync_copy(acc.at[pl.ds(s * NPT + (NPT // CH) * CH, ntl)],
                  msgs.at[pl.ds(0, ntl)])
  pltpu.sync_copy(msgs.at[pl.ds(0, ntl)],
                  out_h.at[c, pl.ds(s * NPT + (NPT // CH) * CH, ntl)])


def _make_layer(width, split_edges):
  mesh = plsc.VectorSubcoreMesh(core_axis_name="c", subcore_axis_name="s")
  return pl.kernel(
      functools.partial(_layer_body, width, split_edges),
      out_type=jax.ShapeDtypeStruct((NC, N, width), f32),
      mesh=mesh,
      compiler_params=pltpu.CompilerParams(use_tc_tiling_on_sc=False),
      scratch_types=[
          pltpu.VMEM((CH,), i32),
          pltpu.VMEM((CH,), i32),
          pltpu.VMEM((CH,), i32),
          pltpu.VMEM((CH,), i32),
          pltpu.VMEM((CH,), i32),
          pltpu.VMEM((CH,), i32),
          pltpu.VMEM((CH,), i32),
          pltpu.VMEM((CH,), i32),
          pltpu.VMEM((CH // SB, SB), i32),
          pltpu.VMEM((CH,), f32),
          pltpu.VMEM((CH, width), f32),
          pltpu.SemaphoreType.DMA,
          pltpu.SemaphoreType.DMA,
          pltpu.SemaphoreType.DMA,
          pltpu.SemaphoreType.DMA,
          pltpu.SemaphoreType.DMA,
          pltpu.VMEM_SHARED((N, width), f32),
      ],
  )


BLK = 2000


def _stage_b_kern(emb_r, w1_r, b1_r, w2_r, b2_r, wcat_r, root1_r, bias1_r,
                  t1_r, r1_r):
  x1 = jax.nn.sigmoid(
      jnp.dot(emb_r[...], w1_r[...], preferred_element_type=f32) + b1_r[...])
  x = jax.nn.sigmoid(
      jnp.dot(x1, w2_r[...], preferred_element_type=f32) + b2_r[...])
  t = jnp.dot(x, wcat_r[...], preferred_element_type=f32)
  t1_r[0] = t[:, :R * 32]
  t1_r[1] = t[:, R * 32:]
  r1_r[...] = jnp.dot(x, root1_r[...], preferred_element_type=f32) + bias1_r[...]


def _inv_kern(c_r, inv_r):
  cnt = c_r[0] + c_r[1]
  inv_r[...] = jnp.where(cnt > 0, 1.0 / jnp.maximum(cnt, 1.0), 0.0)


def _stage_d_kern(agg_r, r1_r, wcat2_r, root2_r, bias2_r, t2_r, r2_r):
  h = jnp.maximum(
      jnp.concatenate([agg_r[0], agg_r[1]], axis=1) + r1_r[...], 0.0)
  t2_r[...] = jnp.dot(h, wcat2_r[...], preferred_element_type=f32)
  r2_r[...] = jnp.dot(h, root2_r[...], preferred_element_type=f32) + bias2_r[...]


def _stage_f_kern(agg_r, r2_r, out_r):
  out_r[...] = jax.nn.sigmoid(agg_r[0] + agg_r[1] + r2_r[...])


def _full(shape):
  return pl.BlockSpec(shape, lambda i: tuple(0 for _ in shape))


def kernel(emb, W1, b1, W2, b2, Wr1, root1, bias1, Wr2, root2, bias2,
           edge_index, edge_type):
  src = edge_index[0]
  dst = edge_index[1]
  et = edge_type

  w1t = W1.T
  w2t = W2.T
  b1r = b1.reshape(1, OUT_F)
  b2r = b2.reshape(1, EMB)
  wlo = Wr1[:, :, :32].transpose(1, 0, 2).reshape(EMB, R * 32)
  whi = Wr1[:, :, 32:].transpose(1, 0, 2).reshape(EMB, R * 32)
  wcat = jnp.concatenate([wlo, whi], axis=1)
  wcat2 = Wr2.transpose(1, 0, 2).reshape(HID, R * LABELS)
  bias1r = bias1.reshape(1, HID)
  bias2r = bias2.reshape(1, LABELS)
  counts = _make_counts()(dst, et)

  inv2d = pl.pallas_call(
      _inv_kern,
      grid=(1,),
      in_specs=[pl.BlockSpec((NC, 3125, 128), lambda i: (0, 0, 0))],
      out_specs=pl.BlockSpec((3125, 128), lambda i: (0, 0)),
      out_shape=jax.ShapeDtypeStruct((3125, 128), f32),
  )(counts.reshape(NC, 3125, 128))
  inv = inv2d.reshape(NR)

  t1, r1 = pl.pallas_call(
      _stage_b_kern,
      grid=(N // BLK,),
      in_specs=[
          pl.BlockSpec((BLK, EMB), lambda i: (i, 0)),
          _full((EMB, OUT_F)),
          _full((1, OUT_F)),
          _full((OUT_F, EMB)),
          _full((1, EMB)),
          _full((EMB, 512)),
          _full((EMB, HID)),
          _full((1, HID)),
      ],
      out_specs=[
          pl.BlockSpec((NC, BLK, R * 32), lambda i: (0, i, 0)),
          pl.BlockSpec((BLK, HID), lambda i: (i, 0)),
      ],
      out_shape=[
          jax.ShapeDtypeStruct((NC, N, R * 32), f32),
          jax.ShapeDtypeStruct((N, HID), f32),
      ],
  )(emb, w1t, b1r, w2t, b2r, wcat, root1, bias1r)

  agg1 = _make_layer(32, False)(
      t1.reshape(NC, NR, 32), inv, src, dst, et)

  t2, r2 = pl.pallas_call(
      _stage_d_kern,
      grid=(N // BLK,),
      in_specs=[
          pl.BlockSpec((NC, BLK, 32), lambda i: (0, i, 0)),
          pl.BlockSpec((BLK, HID), lambda i: (i, 0)),
          _full((HID, R * LABELS)),
          _full((HID, LABELS)),
          _full((1, LABELS)),
      ],
      out_specs=[
          pl.BlockSpec((BLK, R * LABELS), lambda i: (i, 0)),
          pl.BlockSpec((BLK, LABELS), lambda i: (i, 0)),
      ],
      out_shape=[
          jax.ShapeDtypeStruct((N, R * LABELS), f32),
          jax.ShapeDtypeStruct((N, LABELS), f32),
      ],
  )(agg1, r1, wcat2, root2, bias2r)

  agg2 = _make_layer(LABELS, True)(
      t2.reshape(NR, LABELS), inv, src, dst, et)

  out = pl.pallas_call(
      _stage_f_kern,
      grid=(N // BLK,),
      in_specs=[
          pl.BlockSpec((NC, BLK, LABELS), lambda i: (0, i, 0)),
          pl.BlockSpec((BLK, LABELS), lambda i: (i, 0)),
      ],
      out_specs=pl.BlockSpec((BLK, LABELS), lambda i: (i, 0)),
      out_shape=jax.ShapeDtypeStruct((N, LABELS), f32),
  )(agg2, r2)
  return out
